# Initial kernel scaffold; baseline (speedup 1.0000x reference)
import jax
import jax.numpy as jnp
from jax.experimental import pallas as pl


def kernel(x):
    raise NotImplementedError("write your pallas kernel here")


# speedup vs baseline: 1.5500x; 1.5500x over previous
import functools
import numpy as np
import jax
import jax.numpy as jnp
from jax import lax
from jax.experimental import pallas as pl
from jax.experimental.pallas import tpu as pltpu
from jax.experimental.pallas import tpu_sc as plsc

N = 64
LANES = 16
NUM_WORKERS = 32
TRASH = N * N
MAP_WORDS = N * N + LANES
PAD_W = 80


def _bands():
    out = [(0, 1)]
    out += [(d, 1) for d in range(1, 16)]
    out += [(17 + 2 * k, 2) for k in range(8)]
    out += [(35 + 4 * k, 4) for k in range(8)]
    return out


def _build_tables():
    idx_rows = []
    meta = {}
    mask = np.zeros((N, N), dtype=bool)
    off = 0
    for d, s in _bands():
        nv = (N - 1 - d) // LANES + 1
        for t in range(nv):
            row = []
            for l in range(LANES):
                i = LANES * t + l
                if i <= N - 1 - d and i % s == 0:
                    row.append(i * (N + 1) + d)
                    mask[i, i + d] = True
                else:
                    row.append(TRASH + l)
            idx_rows.append(row)
        meta[d] = (nv, off)
        off += nv
    table = np.asarray(idx_rows, dtype=np.int32).reshape(-1)
    return table, meta, mask


_IDX_TABLE, _BAND_META, _MASK2D = _build_tables()
_IDX_WORDS = _IDX_TABLE.shape[0]


def _sc_body(xp_hbm, idx_hbm, outb_hbm, outc_hbm, slab, idxv, mapb, mapc):
    nq = xp_hbm.shape[0] // NUM_WORKERS
    wid = lax.axis_index("s") * 2 + lax.axis_index("c")
    base = wid * nq

    pltpu.sync_copy(xp_hbm.at[pl.ds(base * 1, nq), :], slab)
    pltpu.sync_copy(idx_hbm, idxv)

    zero = jnp.zeros((LANES,), jnp.float32)

    def zbody(i, c):
        mapb[pl.ds(i * LANES, LANES)] = zero
        mapc[pl.ds(i * LANES, LANES)] = zero
        return c

    lax.fori_loop(0, MAP_WORDS // LANES, zbody, 0)

    def qbody(q, c):
        xbase = [slab[q, pl.ds(LANES * t, LANES)] for t in range(4)]
        w = list(xbase)
        nv0, off0 = _BAND_META[0]
        for t in range(nv0):
            iv = idxv[pl.ds((off0 + t) * LANES, LANES)]
            plsc.store_scatter(mapc, [iv], w[t])
            plsc.store_scatter(mapb, [iv], w[t])
        for d in range(1, N):
            tc = (N - 1 - d) // LANES + 1
            xs = [slab[q, pl.ds(d + LANES * t, LANES)] for t in range(tc)]
            for t in range(tc):
                w[t] = jnp.maximum(w[t], xs[t])
            if d in _BAND_META:
                nv, off = _BAND_META[d]
                for t in range(nv):
                    iv = idxv[pl.ds((off + t) * LANES, LANES)]
                    plsc.store_scatter(mapc, [iv], w[t])
                    plsc.store_scatter(mapb, [iv], (xbase[t] + xs[t]) * 0.5)
        pltpu.sync_copy(mapb.at[pl.ds(0, N * N)], outb_hbm.at[base + q])
        pltpu.sync_copy(mapc.at[pl.ds(0, N * N)], outc_hbm.at[base + q])
        return c

    lax.fori_loop(0, nq, qbody, 0)


def kernel(x):
    B, D, n = x.shape
    q_total = B * D
    xp = jnp.pad(x.reshape(q_total, n), ((0, 0), (0, PAD_W - n)))
    idx = jnp.asarray(_IDX_TABLE)

    nq = q_total // NUM_WORKERS
    call = functools.partial(
        pl.kernel,
        mesh=plsc.VectorSubcoreMesh(core_axis_name="c", subcore_axis_name="s"),
        out_type=[
            jax.ShapeDtypeStruct((q_total, N * N), jnp.float32),
            jax.ShapeDtypeStruct((q_total, N * N), jnp.float32),
        ],
        scratch_types=[
            pltpu.VMEM((nq, PAD_W), jnp.float32),
            pltpu.VMEM((_IDX_WORDS,), jnp.int32),
            pltpu.VMEM((MAP_WORDS,), jnp.float32),
            pltpu.VMEM((MAP_WORDS,), jnp.float32),
        ],
        compiler_params=pltpu.CompilerParams(needs_layout_passes=False),
    )
    outb, outc = call(_sc_body)(xp, idx)

    boundary = outb.reshape(B, D, n, n)
    content = outc.reshape(B, D, n, n)
    mask2d = jnp.broadcast_to(jnp.asarray(_MASK2D)[None, None], (B, 1, n, n))
    return (boundary, content, mask2d)


# speedup vs baseline: 1.7425x; 1.1242x over previous
import functools
import numpy as np
import jax
import jax.numpy as jnp
from jax import lax
from jax.experimental import pallas as pl
from jax.experimental.pallas import tpu as pltpu
from jax.experimental.pallas import tpu_sc as plsc

N = 64
LANES = 16
NUM_WORKERS = 32
TRASH = N * N
MAP_WORDS = N * N + LANES
PAD_W = 80


def _bands():
    out = [(0, 1)]
    out += [(d, 1) for d in range(1, 16)]
    out += [(17 + 2 * k, 2) for k in range(8)]
    out += [(35 + 4 * k, 4) for k in range(8)]
    return out


def _build_tables():
    idx_rows = []
    meta = {}
    mask = np.zeros((N, N), dtype=bool)
    off = 0
    for d, s in _bands():
        nv = (N - 1 - d) // LANES + 1
        for t in range(nv):
            row = []
            for l in range(LANES):
                i = LANES * t + l
                if i <= N - 1 - d and i % s == 0:
                    row.append(i * (N + 1) + d)
                    mask[i, i + d] = True
                else:
                    row.append(TRASH + l)
            idx_rows.append(row)
        meta[d] = (nv, off)
        off += nv
    table = np.asarray(idx_rows, dtype=np.int32).reshape(-1)
    return table, meta, mask


_IDX_TABLE, _BAND_META, _MASK2D = _build_tables()
_IDX_WORDS = _IDX_TABLE.shape[0]


def _compute_maps(slab, idxv, q, mapb, mapc):
    xbase = [slab[q, pl.ds(LANES * t, LANES)] for t in range(4)]
    w = list(xbase)
    nv0, off0 = _BAND_META[0]
    for t in range(nv0):
        iv = idxv[pl.ds((off0 + t) * LANES, LANES)]
        plsc.store_scatter(mapc, [iv], w[t])
        plsc.store_scatter(mapb, [iv], w[t])
    for d in range(1, N):
        tc = (N - 1 - d) // LANES + 1
        xs = [slab[q, pl.ds(d + LANES * t, LANES)] for t in range(tc)]
        for t in range(tc):
            w[t] = jnp.maximum(w[t], xs[t])
        if d in _BAND_META:
            nv, off = _BAND_META[d]
            for t in range(nv):
                iv = idxv[pl.ds((off + t) * LANES, LANES)]
                plsc.store_scatter(mapc, [iv], w[t])
                plsc.store_scatter(mapb, [iv], (xbase[t] + xs[t]) * 0.5)


def _sc_body(xp_hbm, idx_hbm, outb_hbm, outc_hbm, slab, idxv, maps_b, maps_c,
             sems):
    nq = xp_hbm.shape[0] // NUM_WORKERS
    wid = lax.axis_index("s") * 2 + lax.axis_index("c")
    base = wid * nq

    pltpu.sync_copy(xp_hbm.at[pl.ds(base * 1, nq), :], slab)
    pltpu.sync_copy(idx_hbm, idxv)

    zero = jnp.zeros((LANES,), jnp.float32)

    def zbody(i, c):
        for b in range(2):
            maps_b[b][pl.ds(i * LANES, LANES)] = zero
            maps_c[b][pl.ds(i * LANES, LANES)] = zero
        return c

    lax.fori_loop(0, MAP_WORDS // LANES, zbody, 0)

    def gbody(g, c):
        for b in range(2):
            q = 2 * g + b

            @pl.when(g > 0)
            def _wait():
                pltpu.make_async_copy(
                    maps_b[b].at[pl.ds(0, N * N)], outb_hbm.at[base + q],
                    sems.at[b]).wait()
                pltpu.make_async_copy(
                    maps_c[b].at[pl.ds(0, N * N)], outc_hbm.at[base + q],
                    sems.at[b]).wait()

            _compute_maps(slab, idxv, q, maps_b[b], maps_c[b])
            pltpu.async_copy(maps_b[b].at[pl.ds(0, N * N)],
                             outb_hbm.at[base + q], sems.at[b])
            pltpu.async_copy(maps_c[b].at[pl.ds(0, N * N)],
                             outc_hbm.at[base + q], sems.at[b])
        return c

    lax.fori_loop(0, nq // 2, gbody, 0)
    for b in range(2):
        q = nq - 2 + b
        pltpu.make_async_copy(maps_b[b].at[pl.ds(0, N * N)],
                              outb_hbm.at[base + q], sems.at[b]).wait()
        pltpu.make_async_copy(maps_c[b].at[pl.ds(0, N * N)],
                              outc_hbm.at[base + q], sems.at[b]).wait()


def kernel(x):
    B, D, n = x.shape
    q_total = B * D
    xp = jnp.pad(x.reshape(q_total, n), ((0, 0), (0, PAD_W - n)))
    idx = jnp.asarray(_IDX_TABLE)

    nq = q_total // NUM_WORKERS
    call = functools.partial(
        pl.kernel,
        mesh=plsc.VectorSubcoreMesh(core_axis_name="c", subcore_axis_name="s"),
        out_type=[
            jax.ShapeDtypeStruct((q_total, N * N), jnp.float32),
            jax.ShapeDtypeStruct((q_total, N * N), jnp.float32),
        ],
        scratch_types=[
            pltpu.VMEM((nq, PAD_W), jnp.float32),
            pltpu.VMEM((_IDX_WORDS,), jnp.int32),
            [pltpu.VMEM((MAP_WORDS,), jnp.float32) for _ in range(2)],
            [pltpu.VMEM((MAP_WORDS,), jnp.float32) for _ in range(2)],
            pltpu.SemaphoreType.DMA((2,)),
        ],
        compiler_params=pltpu.CompilerParams(needs_layout_passes=False),
    )
    outb, outc = call(_sc_body)(xp, idx)

    boundary = outb.reshape(B, D, n, n)
    content = outc.reshape(B, D, n, n)
    mask2d = jnp.broadcast_to(jnp.asarray(_MASK2D)[None, None], (B, 1, n, n))
    return (boundary, content, mask2d)


# speedup vs baseline: 7.1731x; 4.1166x over previous
import functools
import numpy as np
import jax
import jax.numpy as jnp
from jax import lax
from jax.experimental import pallas as pl
from jax.experimental.pallas import tpu as pltpu
from jax.experimental.pallas import tpu_sc as plsc

N = 64
LANES = 16
NUM_WORKERS = 32
HALF = N // 2


def _build_mask():
    mask = np.zeros((N, N), dtype=bool)
    bands = [(0, 1)] + [(d, 1) for d in range(1, 16)]
    bands += [(17 + 2 * k, 2) for k in range(8)]
    bands += [(35 + 4 * k, 4) for k in range(8)]
    for d, s in bands:
        i = np.arange(0, N - d, s)
        mask[i, i + d] = True
    return mask


_MASK2D = _build_mask()
_MASKF = np.zeros(N * N + LANES, dtype=np.float32)
_MASKF[:N * N] = _MASK2D.astype(np.float32).reshape(-1)


def _sc_body(xt_hbm, maskf_hbm, outb_hbm, outc_hbm, xtv, maskv, bufs, sems):
    ndc = xt_hbm.shape[2] // LANES
    wid = lax.axis_index("s") * 2 + lax.axis_index("c")
    b = wid // 4
    r = wid % 4

    pltpu.sync_copy(xt_hbm.at[b], xtv)
    pltpu.sync_copy(maskf_hbm, maskv)

    zero = jnp.zeros((LANES,), jnp.float32)

    def zbody(j, c):
        for buf in bufs:
            for ch in range(ndc):
                buf[j, pl.ds(ch * LANES, LANES)] = zero
        return c

    lax.fori_loop(0, HALF, zbody, 0)

    bl, cl, bh, ch_ = bufs

    def make_jbody(buf_b, buf_c, j0, i, xi):
        def jbody(j, R):
            m = maskv[pl.ds(i * N + j, LANES)][0]
            mh = m * 0.5
            Rn = []
            for c in range(ndc):
                xj = xtv[j, pl.ds(c * LANES, LANES)]
                rc = jnp.maximum(R[c], xj)
                Rn.append(rc)
                buf_c[j - j0, pl.ds(c * LANES, LANES)] = rc * m
                buf_b[j - j0, pl.ds(c * LANES, LANES)] = (xi[c] + xj) * mh
            return tuple(Rn)
        return jbody

    def slab(k, carry):
        i = r + 4 * (15 - k)

        @pl.when(k > 0)
        def _wait_low():
            pltpu.make_async_copy(bl, outb_hbm.at[b, i, pl.ds(0, HALF), :],
                                  sems.at[0]).wait()
            pltpu.make_async_copy(cl, outc_hbm.at[b, i, pl.ds(0, HALF), :],
                                  sems.at[1]).wait()

        xi = [xtv[i, pl.ds(c * LANES, LANES)] for c in range(ndc)]
        R0 = tuple(xi)
        R1 = lax.fori_loop(jnp.minimum(i, HALF), HALF,
                           make_jbody(bl, cl, 0, i, xi), R0)
        pltpu.async_copy(bl, outb_hbm.at[b, i, pl.ds(0, HALF), :], sems.at[0])
        pltpu.async_copy(cl, outc_hbm.at[b, i, pl.ds(0, HALF), :], sems.at[1])

        @pl.when(k > 0)
        def _wait_high():
            pltpu.make_async_copy(bh, outb_hbm.at[b, i, pl.ds(HALF, HALF), :],
                                  sems.at[2]).wait()
            pltpu.make_async_copy(ch_, outc_hbm.at[b, i, pl.ds(HALF, HALF), :],
                                  sems.at[3]).wait()

        lax.fori_loop(jnp.maximum(i, HALF), N,
                      make_jbody(bh, ch_, HALF, i, xi), R1)
        pltpu.async_copy(bh, outb_hbm.at[b, i, pl.ds(HALF, HALF), :],
                         sems.at[2])
        pltpu.async_copy(ch_, outc_hbm.at[b, i, pl.ds(HALF, HALF), :],
                         sems.at[3])
        return carry

    lax.fori_loop(0, 16, slab, 0)

    i_last = r
    pltpu.make_async_copy(bl, outb_hbm.at[b, i_last, pl.ds(0, HALF), :],
                          sems.at[0]).wait()
    pltpu.make_async_copy(cl, outc_hbm.at[b, i_last, pl.ds(0, HALF), :],
                          sems.at[1]).wait()
    pltpu.make_async_copy(bh, outb_hbm.at[b, i_last, pl.ds(HALF, HALF), :],
                          sems.at[2]).wait()
    pltpu.make_async_copy(ch_, outc_hbm.at[b, i_last, pl.ds(HALF, HALF), :],
                          sems.at[3]).wait()


def kernel(x):
    B, D, n = x.shape
    xt = jnp.transpose(x, (0, 2, 1))
    maskf = jnp.asarray(_MASKF)

    call = functools.partial(
        pl.kernel,
        mesh=plsc.VectorSubcoreMesh(core_axis_name="c", subcore_axis_name="s"),
        out_type=[
            jax.ShapeDtypeStruct((B, n, n, D), jnp.float32),
            jax.ShapeDtypeStruct((B, n, n, D), jnp.float32),
        ],
        scratch_types=[
            pltpu.VMEM((n, D), jnp.float32),
            pltpu.VMEM((_MASKF.shape[0],), jnp.float32),
            [pltpu.VMEM((HALF, D), jnp.float32) for _ in range(4)],
            pltpu.SemaphoreType.DMA((4,)),
        ],
        compiler_params=pltpu.CompilerParams(needs_layout_passes=False),
    )
    outb, outc = call(_sc_body)(xt, maskf)

    boundary = jnp.transpose(outb, (0, 3, 1, 2))
    content = jnp.transpose(outc, (0, 3, 1, 2))
    mask2d = jnp.broadcast_to(jnp.asarray(_MASK2D)[None, None], (B, 1, n, n))
    return (boundary, content, mask2d)
